# baseline (speedup 1.0000x reference)
import functools

import jax
import jax.numpy as jnp
from jax import lax
from jax.experimental import pallas as pl
from jax.experimental.pallas import tpu as pltpu
from jax.experimental.pallas import tpu_sc as plsc

NUM_RAAGS = 1000
EMBED_DIM = 128
SEQ_LEN = 512
BATCH = 1024
B_BLK = 16

_INFO = plsc.get_sparse_core_info()
_NC = _INFO.num_cores
_NS = _INFO.num_subcores
_NW = _NC * _NS
_B_PER_W = BATCH // _NW


def _sc_gather(table_hbm, idx_hbm, out_hbm, idx_v, rows_v, sem):
    wid = lax.axis_index("s") * _NC + lax.axis_index("c")
    base = wid * _B_PER_W
    pltpu.sync_copy(idx_hbm.at[pl.ds(base, _B_PER_W)], idx_v)
    pltpu.async_copy(table_hbm.at[idx_v], rows_v, sem).wait()
    pltpu.sync_copy(rows_v, out_hbm.at[pl.ds(base, _B_PER_W)])


def _bcast_kernel(rows_ref, out_ref):
    out_ref[...] = jnp.broadcast_to(rows_ref[...][:, None, :], out_ref.shape)


def kernel(raag_embeddings, table):
    idx = raag_embeddings.reshape(BATCH)

    mesh = plsc.VectorSubcoreMesh(core_axis_name="c", subcore_axis_name="s")
    gathered = pl.kernel(
        _sc_gather,
        mesh=mesh,
        out_type=jax.ShapeDtypeStruct((BATCH, EMBED_DIM), jnp.float32),
        scratch_types=[
            pltpu.VMEM((_B_PER_W,), jnp.int32),
            pltpu.VMEM((_B_PER_W, EMBED_DIM), jnp.float32),
            pltpu.SemaphoreType.DMA,
        ],
    )(table, idx)

    out = pl.pallas_call(
        _bcast_kernel,
        grid=(BATCH // B_BLK,),
        in_specs=[pl.BlockSpec((B_BLK, EMBED_DIM), lambda i: (i, 0))],
        out_specs=pl.BlockSpec((B_BLK, SEQ_LEN, EMBED_DIM), lambda i: (i, 0, 0)),
        out_shape=jax.ShapeDtypeStruct((BATCH, SEQ_LEN, EMBED_DIM), jnp.float32),
    )(gathered)
    return out


# speedup vs baseline: 1.1845x; 1.1845x over previous
import functools

import jax
import jax.numpy as jnp
from jax import lax
from jax.experimental import pallas as pl
from jax.experimental.pallas import tpu as pltpu
from jax.experimental.pallas import tpu_sc as plsc

NUM_RAAGS = 1000
EMBED_DIM = 128
SEQ_LEN = 512
BATCH = 1024
B_BLK = 16

_INFO = plsc.get_sparse_core_info()
_NC = _INFO.num_cores
_NS = _INFO.num_subcores
_NW = _NC * _NS
_B_PER_W = BATCH // _NW


def _sc_gather(table_hbm, idx_hbm, out_hbm, idx_v, rows_v, sem):
    wid = lax.axis_index("s") * _NC + lax.axis_index("c")
    base = wid * _B_PER_W
    pltpu.sync_copy(idx_hbm.at[pl.ds(base, _B_PER_W)], idx_v)
    pltpu.async_copy(table_hbm.at[idx_v], rows_v, sem).wait()
    pltpu.sync_copy(rows_v, out_hbm.at[pl.ds(base, _B_PER_W)])


def _bcast_kernel(rows_ref, out_ref):
    out_ref[...] = jnp.broadcast_to(rows_ref[...][:, None, :], out_ref.shape)


def kernel(raag_embeddings, table):
    idx = raag_embeddings.reshape(BATCH)

    gathered = jnp.take(table, idx, axis=0)

    out = pl.pallas_call(
        _bcast_kernel,
        grid=(BATCH // B_BLK,),
        in_specs=[pl.BlockSpec((B_BLK, EMBED_DIM), lambda i: (i, 0))],
        out_specs=pl.BlockSpec((B_BLK, SEQ_LEN, EMBED_DIM), lambda i: (i, 0, 0)),
        out_shape=jax.ShapeDtypeStruct((BATCH, SEQ_LEN, EMBED_DIM), jnp.float32),
    )(gathered)
    return out
